# baseline (speedup 1.0000x reference)
import functools

import jax
import jax.numpy as jnp
from jax.experimental import pallas as pl
from jax.experimental.pallas import tpu as pltpu

B, S = 1, 2048
D_MODEL = 1024
D_FF = 2048
E = 8
TOP_K = 2
LANES = 128
NF = 4
FF_T = D_FF // NF


def _moe_body(x_ref, xb_ref, wg_ref, bg_ref, w1_ref, b1_ref, w2_ref, b2_ref,
              out_ref, usage_ref, avg_ref, lbl_ref, c_ref):
    e = pl.program_id(0)
    f = pl.program_id(1)
    lane = jax.lax.broadcasted_iota(jnp.int32, (1, LANES), 1)

    @pl.when(jnp.logical_and(e == 0, f == 0))
    def _router():
        x = x_ref[...]
        logits = jax.lax.dot_general(
            x, wg_ref[...], (((1,), (0,)), ((), ())),
            preferred_element_type=jnp.float32) + bg_ref[...]
        m = jnp.max(logits, axis=1, keepdims=True)
        p = jnp.exp(logits - m)
        p = p / jnp.sum(p, axis=1, keepdims=True)
        p1 = jnp.max(p, axis=1, keepdims=True)
        a1 = jnp.min(jnp.where(p == p1, lane, LANES), axis=1, keepdims=True)
        oh1 = (lane == a1).astype(jnp.float32)
        p_m = jnp.where(lane == a1, -1.0, p)
        p2 = jnp.max(p_m, axis=1, keepdims=True)
        a2 = jnp.min(jnp.where(p_m == p2, lane, LANES), axis=1, keepdims=True)
        oh2 = (lane == a2).astype(jnp.float32)
        wsum = p1 + p2
        c_ref[...] = (p1 / wsum) * oh1 + (p2 / wsum) * oh2
        usage_ref[...] = jnp.sum(oh1 + oh2, axis=0, keepdims=True) / (S * TOP_K)
        avg = jnp.sum(p, axis=0, keepdims=True) / S
        avg_ref[...] = avg
        msk = (lane < E).astype(jnp.float32)
        mean = jnp.sum(avg * msk) / E
        var = jnp.sum(msk * (avg - mean) ** 2) / (E - 1)
        lbl_ref[...] = jnp.full((1, LANES), var, dtype=jnp.float32)
        out_ref[...] = jnp.zeros(out_ref.shape, out_ref.dtype)

    sel = (lane == e).astype(jnp.float32)
    c_col = jnp.sum(c_ref[...] * sel, axis=1, keepdims=True)
    h = jax.lax.dot_general(
        xb_ref[...], w1_ref[0], (((1,), (0,)), ((), ())),
        preferred_element_type=jnp.float32) + b1_ref[0]
    h = jnp.maximum(h, 0.0).astype(jnp.bfloat16)
    y = jax.lax.dot_general(
        h, w2_ref[0], (((1,), (0,)), ((), ())),
        preferred_element_type=jnp.float32)

    @pl.when(f == 0)
    def _bias2():
        out_ref[...] += c_col * b2_ref[0]

    out_ref[...] += c_col * y


@functools.partial(jax.jit, static_argnames=())
def kernel(x, W_gate, b_gate, W1, b1, W2, b2):
    x2 = x.reshape(S, D_MODEL)
    wg = jnp.zeros((D_MODEL, LANES), jnp.float32).at[:, :E].set(W_gate)
    bg = jnp.full((1, LANES), -1e30, jnp.float32).at[0, :E].set(b_gate)

    grid = (E, NF)
    out, usage, avg, lbl = pl.pallas_call(
        _moe_body,
        grid=grid,
        in_specs=[
            pl.BlockSpec((S, D_MODEL), lambda e, f: (0, 0)),
            pl.BlockSpec((S, D_MODEL), lambda e, f: (0, 0)),
            pl.BlockSpec((D_MODEL, LANES), lambda e, f: (0, 0)),
            pl.BlockSpec((1, LANES), lambda e, f: (0, 0)),
            pl.BlockSpec((1, D_MODEL, FF_T), lambda e, f: (e, 0, f)),
            pl.BlockSpec((1, 1, FF_T), lambda e, f: (e, 0, f)),
            pl.BlockSpec((1, FF_T, D_MODEL), lambda e, f: (e, f, 0)),
            pl.BlockSpec((1, 1, D_MODEL), lambda e, f: (e, 0, 0)),
        ],
        out_specs=[
            pl.BlockSpec((S, D_MODEL), lambda e, f: (0, 0)),
            pl.BlockSpec((1, LANES), lambda e, f: (0, 0)),
            pl.BlockSpec((1, LANES), lambda e, f: (0, 0)),
            pl.BlockSpec((1, LANES), lambda e, f: (0, 0)),
        ],
        out_shape=[
            jax.ShapeDtypeStruct((S, D_MODEL), jnp.float32),
            jax.ShapeDtypeStruct((1, LANES), jnp.float32),
            jax.ShapeDtypeStruct((1, LANES), jnp.float32),
            jax.ShapeDtypeStruct((1, LANES), jnp.float32),
        ],
        scratch_shapes=[pltpu.VMEM((S, LANES), jnp.float32)],
    )(x2, x2.astype(jnp.bfloat16), wg, bg, W1.astype(jnp.bfloat16),
      b1.reshape(E, 1, D_FF), W2.astype(jnp.bfloat16),
      b2.reshape(E, 1, D_MODEL))

    output = out.reshape(B, S, D_MODEL)
    expert_usage = usage[0, :E]
    load_balance_loss = lbl[0, 0]
    return (output, expert_usage, load_balance_loss)


# speedup vs baseline: 1.4536x; 1.4536x over previous
import functools

import jax
import jax.numpy as jnp
from jax import lax
from jax.experimental import pallas as pl
from jax.experimental.pallas import tpu as pltpu
from jax.experimental.pallas import tpu_sc as plsc

B, S = 1, 2048
D_MODEL = 1024
D_FF = 2048
E = 8
TOP_K = 2
LANES = 128
P = S * TOP_K
BLK = 256
NB = 24
XS_ROWS = NB * BLK
CH = 512
NCH = P // CH
NBPAD = 32
NC, NS, NW = 2, 16, 32
PAIRS_W = P // NW
TOK_W = S // NW


def _router_body(x_ref, wg_ref, bg_ref,
                 pos_ref, w16_ref, eid_ref, usage_ref, lbl_ref):
    lane = lax.broadcasted_iota(jnp.int32, (1, LANES), 1)
    x = x_ref[...]
    logits = lax.dot_general(x, wg_ref[...], (((1,), (0,)), ((), ())),
                             preferred_element_type=jnp.float32) + bg_ref[...]
    m = jnp.max(logits, axis=1, keepdims=True)
    p = jnp.exp(logits - m)
    p = p / jnp.sum(p, axis=1, keepdims=True)
    p1 = jnp.max(p, axis=1, keepdims=True)
    a1 = jnp.min(jnp.where(p == p1, lane, LANES), axis=1, keepdims=True)
    p_m = jnp.where(lane == a1, -1.0, p)
    p2 = jnp.max(p_m, axis=1, keepdims=True)
    a2 = jnp.min(jnp.where(p_m == p2, lane, LANES), axis=1, keepdims=True)
    wsum = p1 + p2
    w_all = jnp.concatenate([p1 / wsum, p2 / wsum], axis=0)
    e_all = jnp.concatenate([a1, a2], axis=0)
    oh = (lane == e_all).astype(jnp.float32)

    ri = lax.broadcasted_iota(jnp.int32, (CH, CH), 0)
    ci = lax.broadcasted_iota(jnp.int32, (CH, CH), 1)
    tri = (ci <= ri).astype(jnp.float32)
    ranks, pres = [], []
    pre = jnp.zeros((1, LANES), jnp.float32)
    for c in range(NCH):
        ohc = oh[c * CH:(c + 1) * CH, :]
        rc = lax.dot_general(tri, ohc, (((1,), (0,)), ((), ())),
                             preferred_element_type=jnp.float32)
        ranks.append(rc)
        pres.append(pre)
        pre = pre + rc[CH - 1:CH, :]
    counts = pre
    padded = jnp.floor((counts + (BLK - 1)) / BLK) * BLK
    lt = (lax.broadcasted_iota(jnp.int32, (LANES, LANES), 0)
          < lax.broadcasted_iota(jnp.int32, (LANES, LANES), 1)).astype(jnp.float32)
    off_pad = lax.dot_general(padded, lt, (((1,), (0,)), ((), ())),
                              preferred_element_type=jnp.float32)
    pos_parts = []
    for c in range(NCH):
        ohc = oh[c * CH:(c + 1) * CH, :]
        vals = jnp.sum(ohc * (off_pad + pres[c] + ranks[c]), axis=1,
                       keepdims=True) - 1.0
        pos_parts.append(vals)
    pos_ref[...] = jnp.concatenate(pos_parts, axis=0).astype(jnp.int32)
    w16_ref[...] = w_all * jnp.ones((1, LANES), jnp.float32)

    seg_end = off_pad + padded
    brow = lax.broadcasted_iota(jnp.int32, (NBPAD, LANES), 0).astype(jnp.float32) * BLK
    valid = (lane < E).astype(jnp.float32)
    eid = jnp.sum((brow >= seg_end).astype(jnp.float32) * valid, axis=1,
                  keepdims=True)
    eid_ref[...] = jnp.minimum(eid, E - 1).astype(jnp.int32)

    usage_ref[...] = counts / P
    avg = jnp.sum(p, axis=0, keepdims=True) / S
    msk = valid
    mean = jnp.sum(avg * msk) / E
    var = jnp.sum(msk * (avg - mean) ** 2) / (E - 1)
    lbl_ref[...] = jnp.full((1, LANES), var, dtype=jnp.float32)


def _router(x2, wg, bg):
    return pl.pallas_call(
        _router_body,
        in_specs=[
            pl.BlockSpec((S, D_MODEL), lambda: (0, 0)),
            pl.BlockSpec((D_MODEL, LANES), lambda: (0, 0)),
            pl.BlockSpec((1, LANES), lambda: (0, 0)),
        ],
        out_specs=[
            pl.BlockSpec((P, 1), lambda: (0, 0)),
            pl.BlockSpec((P, LANES), lambda: (0, 0)),
            pl.BlockSpec((NBPAD, 1), lambda: (0, 0)),
            pl.BlockSpec((1, LANES), lambda: (0, 0)),
            pl.BlockSpec((1, LANES), lambda: (0, 0)),
        ],
        out_shape=[
            jax.ShapeDtypeStruct((P, 1), jnp.int32),
            jax.ShapeDtypeStruct((P, LANES), jnp.float32),
            jax.ShapeDtypeStruct((NBPAD, 1), jnp.int32),
            jax.ShapeDtypeStruct((1, LANES), jnp.float32),
            jax.ShapeDtypeStruct((1, LANES), jnp.float32),
        ],
    )(x2, wg, bg)


_SC_MESH = plsc.VectorSubcoreMesh(core_axis_name="c", subcore_axis_name="s")


@functools.partial(
    pl.kernel, mesh=_SC_MESH,
    out_type=[
        jax.ShapeDtypeStruct((XS_ROWS, D_MODEL), jnp.float32),
        jax.ShapeDtypeStruct((XS_ROWS, LANES), jnp.float32),
    ],
    scratch_types=[
        pltpu.VMEM((PAIRS_W // 16, 16), jnp.int32),
        pltpu.VMEM((16, D_MODEL), jnp.float32),
        pltpu.VMEM((16, LANES), jnp.float32),
        pltpu.SemaphoreType.DMA,
        pltpu.SemaphoreType.DMA,
    ],
)
def _permute(x_hbm, pos_hbm, w16_hbm, xs_hbm, ws_hbm,
             idx_v, xv, wv, sem1, sem2):
    w = lax.axis_index("s") * NC + lax.axis_index("c")
    pltpu.sync_copy(pos_hbm.at[pl.ds(w * (PAIRS_W // 16), PAIRS_W // 16)],
                    idx_v)
    tok_base = lax.rem(w, 16) * PAIRS_W
    for c in range(PAIRS_W // 16):
        pltpu.sync_copy(x_hbm.at[pl.ds(tok_base + c * 16, 16)], xv)
        pltpu.sync_copy(w16_hbm.at[pl.ds(w * PAIRS_W + c * 16, 16)], wv)
        cp1 = pltpu.async_copy(xv, xs_hbm.at[idx_v[c]], sem1)
        cp2 = pltpu.async_copy(wv, ws_hbm.at[idx_v[c]], sem2)
        cp1.wait()
        cp2.wait()


def _ffn_body(eid_ref, xs_ref, w1_ref, b1_ref, w2_ref, b2_ref, ws_ref,
              ys_ref):
    del eid_ref
    h = lax.dot_general(xs_ref[...], w1_ref[0], (((1,), (0,)), ((), ())),
                        preferred_element_type=jnp.float32) + b1_ref[0]
    h = jnp.maximum(h, 0.0)
    y = lax.dot_general(h, w2_ref[0], (((1,), (0,)), ((), ())),
                        preferred_element_type=jnp.float32) + b2_ref[0]
    ys_ref[...] = y * ws_ref[:, 0:1]


def _ffn(eid, xs, W1, b1r, W2, b2r, ws):
    grid_spec = pltpu.PrefetchScalarGridSpec(
        num_scalar_prefetch=1,
        grid=(NB,),
        in_specs=[
            pl.BlockSpec((BLK, D_MODEL), lambda b, eid: (b, 0)),
            pl.BlockSpec((1, D_MODEL, D_FF), lambda b, eid: (eid[b], 0, 0)),
            pl.BlockSpec((1, 1, D_FF), lambda b, eid: (eid[b], 0, 0)),
            pl.BlockSpec((1, D_FF, D_MODEL), lambda b, eid: (eid[b], 0, 0)),
            pl.BlockSpec((1, 1, D_MODEL), lambda b, eid: (eid[b], 0, 0)),
            pl.BlockSpec((BLK, LANES), lambda b, eid: (b, 0)),
        ],
        out_specs=pl.BlockSpec((BLK, D_MODEL), lambda b, eid: (b, 0)),
    )
    return pl.pallas_call(
        _ffn_body,
        grid_spec=grid_spec,
        out_shape=jax.ShapeDtypeStruct((XS_ROWS, D_MODEL), jnp.float32),
    )(eid, xs, W1, b1r, W2, b2r, ws)


@functools.partial(
    pl.kernel, mesh=_SC_MESH,
    out_type=jax.ShapeDtypeStruct((S, D_MODEL), jnp.float32),
    scratch_types=[
        pltpu.VMEM((TOK_W // 16, 16), jnp.int32),
        pltpu.VMEM((TOK_W // 16, 16), jnp.int32),
        pltpu.VMEM((16, D_MODEL), jnp.float32),
        pltpu.VMEM((16, D_MODEL), jnp.float32),
        pltpu.VMEM((16, D_MODEL), jnp.float32),
        pltpu.SemaphoreType.DMA,
        pltpu.SemaphoreType.DMA,
    ],
)
def _combine(ys_hbm, pos_hbm, out_hbm, ia_v, ib_v, ra_v, rb_v, ov,
             sema, semb):
    w = lax.axis_index("s") * NC + lax.axis_index("c")
    ng = TOK_W // 16
    pltpu.sync_copy(pos_hbm.at[pl.ds(w * ng, ng)], ia_v)
    pltpu.sync_copy(pos_hbm.at[pl.ds(S // 16 + w * ng, ng)], ib_v)
    for g in range(ng):
        cpa = pltpu.async_copy(ys_hbm.at[ia_v[g]], ra_v, sema)
        cpb = pltpu.async_copy(ys_hbm.at[ib_v[g]], rb_v, semb)
        cpa.wait()
        cpb.wait()

        def _row(j, carry):
            for cc in range(D_MODEL // 16):
                sl = pl.ds(cc * 16, 16)
                ov[j, sl] = ra_v[j, sl] + rb_v[j, sl]
            return carry

        lax.fori_loop(0, 16, _row, 0)
        pltpu.sync_copy(ov, out_hbm.at[pl.ds(w * TOK_W + g * 16, 16)])


def kernel(x, W_gate, b_gate, W1, b1, W2, b2):
    x2 = x.reshape(S, D_MODEL)
    wg = jnp.zeros((D_MODEL, LANES), jnp.float32).at[:, :E].set(W_gate)
    bg = jnp.full((1, LANES), -1e30, jnp.float32).at[0, :E].set(b_gate)

    pos_col, w16, eidp, usage, lbl = _router(x2, wg, bg)
    pos256 = pos_col.reshape(P // 16, 16)
    eid = eidp[:NB, 0]

    xs, ws = _permute(x2, pos256, w16)
    ys = _ffn(eid, xs, W1, b1.reshape(E, 1, D_FF), W2,
              b2.reshape(E, 1, D_MODEL), ws)
    out2 = _combine(ys, pos256)

    return (out2.reshape(B, S, D_MODEL), usage[0, :E], lbl[0, 0])


# speedup vs baseline: 1.6184x; 1.1134x over previous
import functools

import jax
import jax.numpy as jnp
from jax import lax
from jax.experimental import pallas as pl
from jax.experimental.pallas import tpu as pltpu
from jax.experimental.pallas import tpu_sc as plsc

B, S = 1, 2048
D_MODEL = 1024
D_FF = 2048
E = 8
TOP_K = 2
LANES = 128
P = S * TOP_K
BLK = 256
NB = 24
XS_ROWS = NB * BLK
CH = 512
NCH = P // CH
NBPAD = 32
NC, NS, NW = 2, 16, 32
PAIRS_W = P // NW
TOK_W = S // NW


def _router_body(x_ref, wg_ref, bg_ref,
                 pos_ref, w16_ref, eid_ref, usage_ref, lbl_ref):
    lane = lax.broadcasted_iota(jnp.int32, (1, LANES), 1)
    x = x_ref[...]
    logits = lax.dot_general(x, wg_ref[...], (((1,), (0,)), ((), ())),
                             preferred_element_type=jnp.float32) + bg_ref[...]
    m = jnp.max(logits, axis=1, keepdims=True)
    p = jnp.exp(logits - m)
    p = p / jnp.sum(p, axis=1, keepdims=True)
    p1 = jnp.max(p, axis=1, keepdims=True)
    a1 = jnp.min(jnp.where(p == p1, lane, LANES), axis=1, keepdims=True)
    p_m = jnp.where(lane == a1, -1.0, p)
    p2 = jnp.max(p_m, axis=1, keepdims=True)
    a2 = jnp.min(jnp.where(p_m == p2, lane, LANES), axis=1, keepdims=True)
    wsum = p1 + p2
    w_all = jnp.concatenate([p1 / wsum, p2 / wsum], axis=0)
    e_all = jnp.concatenate([a1, a2], axis=0)
    oh = (lane == e_all).astype(jnp.float32)

    ri = lax.broadcasted_iota(jnp.int32, (CH, CH), 0)
    ci = lax.broadcasted_iota(jnp.int32, (CH, CH), 1)
    tri = (ci <= ri).astype(jnp.float32)
    ranks, pres = [], []
    pre = jnp.zeros((1, LANES), jnp.float32)
    for c in range(NCH):
        ohc = oh[c * CH:(c + 1) * CH, :]
        rc = lax.dot_general(tri, ohc, (((1,), (0,)), ((), ())),
                             preferred_element_type=jnp.float32)
        ranks.append(rc)
        pres.append(pre)
        pre = pre + rc[CH - 1:CH, :]
    counts = pre
    padded = jnp.floor((counts + (BLK - 1)) / BLK) * BLK
    lt = (lax.broadcasted_iota(jnp.int32, (LANES, LANES), 0)
          < lax.broadcasted_iota(jnp.int32, (LANES, LANES), 1)).astype(jnp.float32)
    off_pad = lax.dot_general(padded, lt, (((1,), (0,)), ((), ())),
                              preferred_element_type=jnp.float32)
    pos_parts = []
    for c in range(NCH):
        ohc = oh[c * CH:(c + 1) * CH, :]
        vals = jnp.sum(ohc * (off_pad + pres[c] + ranks[c]), axis=1,
                       keepdims=True) - 1.0
        pos_parts.append(vals)
    pos_ref[...] = jnp.concatenate(pos_parts, axis=0).astype(jnp.int32)
    w16_ref[...] = w_all * jnp.ones((1, LANES), jnp.float32)

    seg_end = off_pad + padded
    brow = lax.broadcasted_iota(jnp.int32, (NBPAD, LANES), 0).astype(jnp.float32) * BLK
    valid = (lane < E).astype(jnp.float32)
    eid = jnp.sum((brow >= seg_end).astype(jnp.float32) * valid, axis=1,
                  keepdims=True)
    eid = jnp.minimum(eid, E - 1)
    nused = jnp.sum(padded * valid) / BLK
    rowi = lax.broadcasted_iota(jnp.int32, (NBPAD, 1), 0)
    eid_ref[...] = jnp.where(rowi == NBPAD - 1, nused, eid).astype(jnp.int32)

    usage_ref[...] = counts / P
    avg = jnp.sum(p, axis=0, keepdims=True) / S
    msk = valid
    mean = jnp.sum(avg * msk) / E
    var = jnp.sum(msk * (avg - mean) ** 2) / (E - 1)
    lbl_ref[...] = jnp.full((1, LANES), var, dtype=jnp.float32)


def _router(x2, wg, bg):
    return pl.pallas_call(
        _router_body,
        in_specs=[
            pl.BlockSpec((S, D_MODEL), lambda: (0, 0)),
            pl.BlockSpec((D_MODEL, LANES), lambda: (0, 0)),
            pl.BlockSpec((1, LANES), lambda: (0, 0)),
        ],
        out_specs=[
            pl.BlockSpec((P, 1), lambda: (0, 0)),
            pl.BlockSpec((P, LANES), lambda: (0, 0)),
            pl.BlockSpec((NBPAD, 1), lambda: (0, 0)),
            pl.BlockSpec((1, LANES), lambda: (0, 0)),
            pl.BlockSpec((1, LANES), lambda: (0, 0)),
        ],
        out_shape=[
            jax.ShapeDtypeStruct((P, 1), jnp.int32),
            jax.ShapeDtypeStruct((P, LANES), jnp.float32),
            jax.ShapeDtypeStruct((NBPAD, 1), jnp.int32),
            jax.ShapeDtypeStruct((1, LANES), jnp.float32),
            jax.ShapeDtypeStruct((1, LANES), jnp.float32),
        ],
    )(x2, wg, bg)


_SC_MESH = plsc.VectorSubcoreMesh(core_axis_name="c", subcore_axis_name="s")


@functools.partial(
    pl.kernel, mesh=_SC_MESH,
    out_type=[
        jax.ShapeDtypeStruct((XS_ROWS, D_MODEL), jnp.float32),
        jax.ShapeDtypeStruct((XS_ROWS, LANES), jnp.float32),
    ],
    scratch_types=[
        pltpu.VMEM((PAIRS_W // 16, 16), jnp.int32),
        pltpu.VMEM((6, 16, D_MODEL), jnp.float32),
        pltpu.VMEM((6, 16, LANES), jnp.float32),
        pltpu.SemaphoreType.DMA,
        pltpu.SemaphoreType.DMA,
        pltpu.SemaphoreType.DMA,
    ],
)
def _permute(x_hbm, pos_hbm, w16_hbm, xs_hbm, ws_hbm,
             idx_v, xv, wv, seml, sem1, sem2):
    w = lax.axis_index("s") * NC + lax.axis_index("c")
    nch = PAIRS_W // 16
    nbuf = 6
    pltpu.sync_copy(pos_hbm.at[pl.ds(w * nch, nch)], idx_v)
    tok_base = lax.rem(w, 16) * PAIRS_W
    loads, scats = {}, {}

    def _start_load(c):
        b = c % nbuf
        loads[c] = (
            pltpu.async_copy(x_hbm.at[pl.ds(tok_base + c * 16, 16)],
                             xv.at[b], seml),
            pltpu.async_copy(w16_hbm.at[pl.ds(w * PAIRS_W + c * 16, 16)],
                             wv.at[b], seml),
        )

    for c in range(min(nbuf, nch)):
        _start_load(c)
    for c in range(nch):
        b = c % nbuf
        loads[c][0].wait()
        loads[c][1].wait()
        scats[c] = (
            pltpu.async_copy(xv.at[b], xs_hbm.at[idx_v[c]], sem1),
            pltpu.async_copy(wv.at[b], ws_hbm.at[idx_v[c]], sem2),
        )
        nxt = c + nbuf
        if nxt < nch:
            scats[c][0].wait()
            scats[c][1].wait()
            del scats[c]
            _start_load(nxt)
    for c in sorted(scats):
        scats[c][0].wait()
        scats[c][1].wait()


def _ffn_body(eid_ref, xs_ref, w1_ref, b1_ref, w2_ref, b2_ref, ws_ref,
              ys_ref):
    @pl.when(pl.program_id(0) < eid_ref[NBPAD - 1, 0])
    def _compute():
        h = lax.dot_general(xs_ref[...], w1_ref[0], (((1,), (0,)), ((), ())),
                            preferred_element_type=jnp.float32) + b1_ref[0]
        h = jnp.maximum(h, 0.0)
        y = lax.dot_general(h, w2_ref[0], (((1,), (0,)), ((), ())),
                            preferred_element_type=jnp.float32) + b2_ref[0]
        ys_ref[...] = y * ws_ref[:, 0:1]


def _ffn(eid, xs, W1, b1r, W2, b2r, ws):
    grid_spec = pltpu.PrefetchScalarGridSpec(
        num_scalar_prefetch=1,
        grid=(NB,),
        in_specs=[
            pl.BlockSpec((BLK, D_MODEL), lambda b, eid: (b, 0)),
            pl.BlockSpec((1, D_MODEL, D_FF), lambda b, eid: (eid[b, 0], 0, 0)),
            pl.BlockSpec((1, 1, D_FF), lambda b, eid: (eid[b, 0], 0, 0)),
            pl.BlockSpec((1, D_FF, D_MODEL), lambda b, eid: (eid[b, 0], 0, 0)),
            pl.BlockSpec((1, 1, D_MODEL), lambda b, eid: (eid[b, 0], 0, 0)),
            pl.BlockSpec((BLK, LANES), lambda b, eid: (b, 0)),
        ],
        out_specs=pl.BlockSpec((BLK, D_MODEL), lambda b, eid: (b, 0)),
    )
    return pl.pallas_call(
        _ffn_body,
        grid_spec=grid_spec,
        out_shape=jax.ShapeDtypeStruct((XS_ROWS, D_MODEL), jnp.float32),
    )(eid, xs, W1, b1r, W2, b2r, ws)


@functools.partial(
    pl.kernel, mesh=_SC_MESH,
    out_type=jax.ShapeDtypeStruct((S, D_MODEL), jnp.float32),
    scratch_types=[
        pltpu.VMEM((TOK_W // 16, 16), jnp.int32),
        pltpu.VMEM((TOK_W // 16, 16), jnp.int32),
        pltpu.VMEM((2, 16, D_MODEL), jnp.float32),
        pltpu.VMEM((2, 16, D_MODEL), jnp.float32),
        pltpu.VMEM((2, 16, D_MODEL), jnp.float32),
        pltpu.SemaphoreType.DMA,
        pltpu.SemaphoreType.DMA,
        pltpu.SemaphoreType.DMA,
    ],
)
def _combine(ys_hbm, pos_hbm, out_hbm, ia_v, ib_v, ra_v, rb_v, ov,
             sema, semb, semo):
    w = lax.axis_index("s") * NC + lax.axis_index("c")
    ng = TOK_W // 16
    pltpu.sync_copy(pos_hbm.at[pl.ds(w * ng, ng)], ia_v)
    pltpu.sync_copy(pos_hbm.at[pl.ds(S // 16 + w * ng, ng)], ib_v)
    gets, puts = {}, {}

    def _start_get(g):
        b = g % 2
        gets[g] = (
            pltpu.async_copy(ys_hbm.at[ia_v[g]], ra_v.at[b], sema),
            pltpu.async_copy(ys_hbm.at[ib_v[g]], rb_v.at[b], semb),
        )

    _start_get(0)
    _start_get(1)
    for g in range(ng):
        b = g % 2
        gets[g][0].wait()
        gets[g][1].wait()
        if g >= 2:
            puts[g - 2].wait()

        def _row(j, carry):
            for cc in range(D_MODEL // 16):
                sl = pl.ds(cc * 16, 16)
                ov[b, j, sl] = ra_v[b, j, sl] + rb_v[b, j, sl]
            return carry

        lax.fori_loop(0, 16, _row, 0)
        puts[g] = pltpu.async_copy(
            ov.at[b], out_hbm.at[pl.ds(w * TOK_W + g * 16, 16)], semo)
        if g + 2 < ng:
            _start_get(g + 2)
    puts[ng - 2].wait()
    puts[ng - 1].wait()


def kernel(x, W_gate, b_gate, W1, b1, W2, b2):
    x2 = x.reshape(S, D_MODEL)
    wg = jnp.zeros((D_MODEL, LANES), jnp.float32).at[:, :E].set(W_gate)
    bg = jnp.full((1, LANES), -1e30, jnp.float32).at[0, :E].set(b_gate)

    pos_col, w16, eidp, usage, lbl = _router(x2, wg, bg)
    pos256 = pos_col.reshape(P // 16, 16)
    eid = eidp

    xs, ws = _permute(x2, pos256, w16)
    ys = _ffn(eid, xs, W1, b1.reshape(E, 1, D_FF), W2,
              b2.reshape(E, 1, D_MODEL), ws)
    out2 = _combine(ys, pos256)

    return (out2.reshape(B, S, D_MODEL), usage[0, :E], lbl[0, 0])


# speedup vs baseline: 1.6491x; 1.0190x over previous
import functools

import jax
import jax.numpy as jnp
from jax import lax
from jax.experimental import pallas as pl
from jax.experimental.pallas import tpu as pltpu
from jax.experimental.pallas import tpu_sc as plsc

B, S = 1, 2048
D_MODEL = 1024
D_FF = 2048
E = 8
TOP_K = 2
LANES = 128
P = S * TOP_K
BLK = 256
NB = 24
XS_ROWS = NB * BLK
CH = 512
NCH = P // CH
NBPAD = 32
NC, NS, NW = 2, 16, 32
PAIRS_W = P // NW
TOK_W = S // NW


def _router_body(x_ref, wg_ref, bg_ref,
                 pos_ref, w16_ref, eid_ref, run_ref, usage_ref, lbl_ref):
    lane = lax.broadcasted_iota(jnp.int32, (1, LANES), 1)
    x = x_ref[...]
    logits = lax.dot_general(x, wg_ref[...], (((1,), (0,)), ((), ())),
                             preferred_element_type=jnp.float32) + bg_ref[...]
    m = jnp.max(logits, axis=1, keepdims=True)
    p = jnp.exp(logits - m)
    p = p / jnp.sum(p, axis=1, keepdims=True)
    p1 = jnp.max(p, axis=1, keepdims=True)
    a1 = jnp.min(jnp.where(p == p1, lane, LANES), axis=1, keepdims=True)
    p_m = jnp.where(lane == a1, -1.0, p)
    p2 = jnp.max(p_m, axis=1, keepdims=True)
    a2 = jnp.min(jnp.where(p_m == p2, lane, LANES), axis=1, keepdims=True)
    wsum = p1 + p2
    w_all = jnp.concatenate([p1 / wsum, p2 / wsum], axis=0)
    e_all = jnp.concatenate([a1, a2], axis=0)
    oh = (lane == e_all).astype(jnp.float32)

    ri = lax.broadcasted_iota(jnp.int32, (CH, CH), 0)
    ci = lax.broadcasted_iota(jnp.int32, (CH, CH), 1)
    tri = (ci <= ri).astype(jnp.float32)
    ranks, pres = [], []
    pre = jnp.zeros((1, LANES), jnp.float32)
    for c in range(NCH):
        ohc = oh[c * CH:(c + 1) * CH, :]
        rc = lax.dot_general(tri, ohc, (((1,), (0,)), ((), ())),
                             preferred_element_type=jnp.float32)
        ranks.append(rc)
        pres.append(pre)
        pre = pre + rc[CH - 1:CH, :]
    counts = pre
    padded = jnp.floor((counts + (BLK - 1)) / BLK) * BLK
    lt = (lax.broadcasted_iota(jnp.int32, (LANES, LANES), 0)
          < lax.broadcasted_iota(jnp.int32, (LANES, LANES), 1)).astype(jnp.float32)
    off_pad = lax.dot_general(padded, lt, (((1,), (0,)), ((), ())),
                              preferred_element_type=jnp.float32)
    pos_parts = []
    for c in range(NCH):
        ohc = oh[c * CH:(c + 1) * CH, :]
        vals = jnp.sum(ohc * (off_pad + pres[c] + ranks[c]), axis=1,
                       keepdims=True) - 1.0
        pos_parts.append(vals)
    pos_ref[...] = jnp.concatenate(pos_parts, axis=0).astype(jnp.int32)
    w16_ref[...] = w_all * jnp.ones((1, LANES), jnp.float32)

    seg_end = off_pad + padded
    brow = lax.broadcasted_iota(jnp.int32, (NBPAD, LANES), 0).astype(jnp.float32) * BLK
    valid = (lane < E).astype(jnp.float32)
    eid = jnp.sum((brow >= seg_end).astype(jnp.float32) * valid, axis=1,
                  keepdims=True)
    eid = jnp.minimum(eid, E - 1)
    nused = jnp.sum(padded * valid) / BLK
    rowi = lax.broadcasted_iota(jnp.int32, (NBPAD, 1), 0)
    eid_ref[...] = jnp.where(rowi == NBPAD - 1, nused, eid).astype(jnp.int32)

    present = (padded > 0).astype(jnp.float32)
    le = (lax.broadcasted_iota(jnp.int32, (LANES, LANES), 0)
          <= lax.broadcasted_iota(jnp.int32, (LANES, LANES), 1)).astype(jnp.float32)
    rank = lax.dot_general(present, le, (((1,), (0,)), ((), ())),
                           preferred_element_type=jnp.float32) - 1.0
    ohb = (jnp.abs(lane.astype(jnp.float32) - eid) < 0.5).astype(jnp.float32)
    run_of = jnp.sum(ohb * rank, axis=1, keepdims=True)
    rtarget = (rowi - (NBPAD - E)).astype(jnp.float32)
    rmatch = (jnp.abs(rank - rtarget) < 0.5) * present
    run_e = jnp.sum(rmatch * lane.astype(jnp.float32), axis=1, keepdims=True)
    run_ref[...] = jnp.where(rowi < NB, run_of, run_e).astype(jnp.int32)

    usage_ref[...] = counts / P
    avg = jnp.sum(p, axis=0, keepdims=True) / S
    msk = valid
    mean = jnp.sum(avg * msk) / E
    var = jnp.sum(msk * (avg - mean) ** 2) / (E - 1)
    lbl_ref[...] = jnp.full((1, LANES), var, dtype=jnp.float32)


def _router(x2, wg, bg):
    return pl.pallas_call(
        _router_body,
        in_specs=[
            pl.BlockSpec((S, D_MODEL), lambda: (0, 0)),
            pl.BlockSpec((D_MODEL, LANES), lambda: (0, 0)),
            pl.BlockSpec((1, LANES), lambda: (0, 0)),
        ],
        out_specs=[
            pl.BlockSpec((P, 1), lambda: (0, 0)),
            pl.BlockSpec((P, LANES), lambda: (0, 0)),
            pl.BlockSpec((NBPAD, 1), lambda: (0, 0)),
            pl.BlockSpec((NBPAD, 1), lambda: (0, 0)),
            pl.BlockSpec((1, LANES), lambda: (0, 0)),
            pl.BlockSpec((1, LANES), lambda: (0, 0)),
        ],
        out_shape=[
            jax.ShapeDtypeStruct((P, 1), jnp.int32),
            jax.ShapeDtypeStruct((P, LANES), jnp.float32),
            jax.ShapeDtypeStruct((NBPAD, 1), jnp.int32),
            jax.ShapeDtypeStruct((NBPAD, 1), jnp.int32),
            jax.ShapeDtypeStruct((1, LANES), jnp.float32),
            jax.ShapeDtypeStruct((1, LANES), jnp.float32),
        ],
    )(x2, wg, bg)


_SC_MESH = plsc.VectorSubcoreMesh(core_axis_name="c", subcore_axis_name="s")


@functools.partial(
    pl.kernel, mesh=_SC_MESH,
    out_type=[
        jax.ShapeDtypeStruct((XS_ROWS, D_MODEL), jnp.float32),
        jax.ShapeDtypeStruct((XS_ROWS, LANES), jnp.float32),
    ],
    scratch_types=[
        pltpu.VMEM((PAIRS_W // 16, 16), jnp.int32),
        pltpu.VMEM((6, 16, D_MODEL), jnp.float32),
        pltpu.VMEM((6, 16, LANES), jnp.float32),
        pltpu.SemaphoreType.DMA,
        pltpu.SemaphoreType.DMA,
        pltpu.SemaphoreType.DMA,
    ],
)
def _permute(x_hbm, pos_hbm, w16_hbm, xs_hbm, ws_hbm,
             idx_v, xv, wv, seml, sem1, sem2):
    w = lax.axis_index("s") * NC + lax.axis_index("c")
    nch = PAIRS_W // 16
    nbuf = 6
    pltpu.sync_copy(pos_hbm.at[pl.ds(w * nch, nch)], idx_v)
    tok_base = lax.rem(w, 16) * PAIRS_W
    loads, scats = {}, {}

    def _start_load(c):
        b = c % nbuf
        loads[c] = (
            pltpu.async_copy(x_hbm.at[pl.ds(tok_base + c * 16, 16)],
                             xv.at[b], seml),
            pltpu.async_copy(w16_hbm.at[pl.ds(w * PAIRS_W + c * 16, 16)],
                             wv.at[b], seml),
        )

    for c in range(min(nbuf, nch)):
        _start_load(c)
    for c in range(nch):
        b = c % nbuf
        loads[c][0].wait()
        loads[c][1].wait()
        scats[c] = (
            pltpu.async_copy(xv.at[b], xs_hbm.at[idx_v[c]], sem1),
            pltpu.async_copy(wv.at[b], ws_hbm.at[idx_v[c]], sem2),
        )
        nxt = c + nbuf
        if nxt < nch:
            scats[c][0].wait()
            scats[c][1].wait()
            del scats[c]
            _start_load(nxt)
    for c in sorted(scats):
        scats[c][0].wait()
        scats[c][1].wait()


def _ffn_body(eid_ref, run_ref, xs_ref, w1_hbm, b1_ref, w2_hbm, b2_ref,
              ws_ref, ys_ref, w1b, w2b, sems):
    b = pl.program_id(0)
    r = run_ref[b, 0]
    slot = lax.rem(r, 2)
    rprev = run_ref[jnp.maximum(b - 1, 0), 0]
    rnext = run_ref[jnp.minimum(b + 1, NB - 1), 0]
    rmax = run_ref[NB - 1, 0]
    is_first = jnp.logical_or(b == 0, rprev != r)
    is_last = jnp.logical_or(b == NB - 1, rnext != r)

    def _load(rr, sl):
        e = run_ref[NBPAD - E + jnp.minimum(rr, E - 1), 0]
        pltpu.make_async_copy(w1_hbm.at[e], w1b.at[sl], sems.at[sl]).start()
        pltpu.make_async_copy(w2_hbm.at[e], w2b.at[sl], sems.at[sl]).start()

    @pl.when(b == 0)
    def _prologue():
        _load(0, 0)

        @pl.when(rmax >= 1)
        def _second():
            _load(1, 1)

    @pl.when(is_first)
    def _wait_weights():
        pltpu.make_async_copy(w1_hbm.at[0], w1b.at[slot], sems.at[slot]).wait()
        pltpu.make_async_copy(w2_hbm.at[0], w2b.at[slot], sems.at[slot]).wait()

    @pl.when(b < eid_ref[NBPAD - 1, 0])
    def _compute():
        h = lax.dot_general(xs_ref[...], w1b[slot], (((1,), (0,)), ((), ())),
                            preferred_element_type=jnp.float32) + b1_ref[0]
        h = jnp.maximum(h, 0.0)
        y = lax.dot_general(h, w2b[slot], (((1,), (0,)), ((), ())),
                            preferred_element_type=jnp.float32) + b2_ref[0]
        ys_ref[...] = y * ws_ref[:, 0:1]

    @pl.when(jnp.logical_and(is_last, r + 2 <= rmax))
    def _prefetch_next_run():
        _load(r + 2, slot)


def _ffn(eid, run, xs, W1, b1r, W2, b2r, ws):
    grid_spec = pltpu.PrefetchScalarGridSpec(
        num_scalar_prefetch=2,
        grid=(NB,),
        in_specs=[
            pl.BlockSpec((BLK, D_MODEL), lambda b, eid, run: (b, 0)),
            pl.BlockSpec(memory_space=pl.ANY),
            pl.BlockSpec((1, 1, D_FF), lambda b, eid, run: (eid[b, 0], 0, 0)),
            pl.BlockSpec(memory_space=pl.ANY),
            pl.BlockSpec((1, 1, D_MODEL), lambda b, eid, run: (eid[b, 0], 0, 0)),
            pl.BlockSpec((BLK, LANES), lambda b, eid, run: (b, 0)),
        ],
        out_specs=pl.BlockSpec((BLK, D_MODEL), lambda b, eid, run: (b, 0)),
        scratch_shapes=[
            pltpu.VMEM((2, D_MODEL, D_FF), jnp.float32),
            pltpu.VMEM((2, D_FF, D_MODEL), jnp.float32),
            pltpu.SemaphoreType.DMA((2,)),
        ],
    )
    return pl.pallas_call(
        _ffn_body,
        grid_spec=grid_spec,
        out_shape=jax.ShapeDtypeStruct((XS_ROWS, D_MODEL), jnp.float32),
    )(eid, run, xs, W1, b1r, W2, b2r, ws)


@functools.partial(
    pl.kernel, mesh=_SC_MESH,
    out_type=jax.ShapeDtypeStruct((S, D_MODEL), jnp.float32),
    scratch_types=[
        pltpu.VMEM((TOK_W // 16, 16), jnp.int32),
        pltpu.VMEM((TOK_W // 16, 16), jnp.int32),
        pltpu.VMEM((2, 16, D_MODEL), jnp.float32),
        pltpu.VMEM((2, 16, D_MODEL), jnp.float32),
        pltpu.VMEM((2, 16, D_MODEL), jnp.float32),
        pltpu.SemaphoreType.DMA,
        pltpu.SemaphoreType.DMA,
        pltpu.SemaphoreType.DMA,
    ],
)
def _combine(ys_hbm, pos_hbm, out_hbm, ia_v, ib_v, ra_v, rb_v, ov,
             sema, semb, semo):
    w = lax.axis_index("s") * NC + lax.axis_index("c")
    ng = TOK_W // 16
    pltpu.sync_copy(pos_hbm.at[pl.ds(w * ng, ng)], ia_v)
    pltpu.sync_copy(pos_hbm.at[pl.ds(S // 16 + w * ng, ng)], ib_v)
    gets, puts = {}, {}

    def _start_get(g):
        b = g % 2
        gets[g] = (
            pltpu.async_copy(ys_hbm.at[ia_v[g]], ra_v.at[b], sema),
            pltpu.async_copy(ys_hbm.at[ib_v[g]], rb_v.at[b], semb),
        )

    _start_get(0)
    _start_get(1)
    for g in range(ng):
        b = g % 2
        gets[g][0].wait()
        gets[g][1].wait()
        if g >= 2:
            puts[g - 2].wait()

        def _row(j, carry):
            for cc in range(D_MODEL // 16):
                sl = pl.ds(cc * 16, 16)
                ov[b, j, sl] = ra_v[b, j, sl] + rb_v[b, j, sl]
            return carry

        lax.fori_loop(0, 16, _row, 0)
        puts[g] = pltpu.async_copy(
            ov.at[b], out_hbm.at[pl.ds(w * TOK_W + g * 16, 16)], semo)
        if g + 2 < ng:
            _start_get(g + 2)
    puts[ng - 2].wait()
    puts[ng - 1].wait()


def kernel(x, W_gate, b_gate, W1, b1, W2, b2):
    x2 = x.reshape(S, D_MODEL)
    wg = jnp.zeros((D_MODEL, LANES), jnp.float32).at[:, :E].set(W_gate)
    bg = jnp.full((1, LANES), -1e30, jnp.float32).at[0, :E].set(b_gate)

    pos_col, w16, eidp, runp, usage, lbl = _router(x2, wg, bg)
    pos256 = pos_col.reshape(P // 16, 16)

    xs, ws = _permute(x2, pos256, w16)
    ys = _ffn(eidp, runp, xs, W1, b1.reshape(E, 1, D_FF), W2,
              b2.reshape(E, 1, D_MODEL), ws)
    out2 = _combine(ys, pos256)

    return (out2.reshape(B, S, D_MODEL), usage[0, :E], lbl[0, 0])
